# Initial kernel scaffold; baseline (speedup 1.0000x reference)
import jax
import jax.numpy as jnp
from jax.experimental import pallas as pl


def kernel(x, in_proj_w, conv1d_w, conv1d_b, x_proj_w, dt_proj_w, dt_proj_b, A_log, Dp, out_proj_w, pw_w, pw_b, bn_gamma, bn_beta, bn_mean, bn_var, gate_w, gate_b):
    raise NotImplementedError("write your pallas kernel here")


# speedup vs baseline: 79.1944x; 79.1944x over previous
import jax
import jax.numpy as jnp
from jax import lax
from jax.experimental import pallas as pl
from jax.experimental.pallas import tpu as pltpu

D_STATE = 16
DT_RANK = 4
BN_EPS = 1e-5

_F32 = jnp.float32
_DIMS_NT = (((1,), (1,)), ((), ()))
_DIMS_NN = (((1,), (0,)), ((), ()))


def _dot_nt(a, b):
    return lax.dot_general(a, b, _DIMS_NT, preferred_element_type=_F32)


def _dot_nn(a, b):
    return lax.dot_general(a, b, _DIMS_NN, preferred_element_type=_F32)


def _block(seq_ref, wi_ref, cw_ref, cb_ref, dtw_ref, dtb_ref, bcw_ref,
           at_ref, dp_ref, wo_ref, pw_ref, bns_ref, bnb_ref, gw_ref, gb_ref,
           skip_ref, down_ref, dt_s, du_s, bct_s, ys_s):
    L, C = seq_ref.shape[1], seq_ref.shape[2]
    Di = dp_ref.shape[1]
    Wd = 64
    seq = seq_ref[0]

    xz = _dot_nt(seq, wi_ref[...])
    u0 = xz[:, :Di]
    z = xz[:, Di:]

    um1 = jnp.concatenate([jnp.zeros((1, Di), _F32), u0[:-1]], axis=0)
    um2 = jnp.concatenate([jnp.zeros((2, Di), _F32), u0[:-2]], axis=0)
    uc = (u0 * cw_ref[2:3, :] + um1 * cw_ref[1:2, :] + um2 * cw_ref[0:1, :]
          + cb_ref[...])
    u = uc * jax.nn.sigmoid(uc)

    dt = jax.nn.softplus(_dot_nn(u, dtw_ref[...]) + dtb_ref[...])
    bct = _dot_nt(bcw_ref[...], u)

    dt_s[...] = dt
    du_s[...] = dt * u
    bct_s[...] = bct

    at = at_ref[...]
    TB = 128

    def body(i, h):
        t0 = pl.multiple_of(i * TB, TB)
        dtb = dt_s[pl.ds(t0, TB), :]
        dub = du_s[pl.ds(t0, TB), :]
        bcb = bct_s[:, pl.ds(t0, TB)]
        for j in range(TB):
            dA = jnp.exp(dtb[j:j + 1, :] * at)
            bc = jnp.broadcast_to(bcb[0:D_STATE, j:j + 1], (D_STATE, Di))
            cc = jnp.broadcast_to(
                bcb[D_STATE:2 * D_STATE, j:j + 1], (D_STATE, Di))
            h = dA * h + dub[j:j + 1, :] * bc
            ys_s[pl.ds(t0 + j, 1), :, :] = jnp.sum(
                h * cc, axis=0, keepdims=True)[:, None, :]
        return h

    lax.fori_loop(0, L // TB, body, jnp.zeros((D_STATE, Di), _F32))

    ys = ys_s[...].reshape(L, Di)
    y = (ys + dp_ref[...] * u) * (z * jax.nn.sigmoid(z))
    res = seq + _dot_nt(y, wo_ref[...])

    col = lax.broadcasted_iota(jnp.int32, (L, 1), 0) % Wd
    acc = None
    k = 0
    for ky in (-1, 0, 1):
        for kx in (-1, 0, 1):
            s = ky * Wd + kx
            if s > 0:
                tap = jnp.concatenate(
                    [res[s:], jnp.zeros((s, C), _F32)], axis=0)
            elif s < 0:
                tap = jnp.concatenate(
                    [jnp.zeros((-s, C), _F32), res[:s]], axis=0)
            else:
                tap = res
            if kx == 1:
                tap = jnp.where(col != Wd - 1, tap, 0.0)
            elif kx == -1:
                tap = jnp.where(col != 0, tap, 0.0)
            part = _dot_nt(tap, pw_ref[k])
            acc = part if acc is None else acc + part
            k += 1

    skip_hwc = jnp.maximum(acc * bns_ref[...] + bnb_ref[...], 0.0)
    skip_ref[0] = jnp.transpose(skip_hwc)

    Co = skip_hwc.shape[1]
    p1 = skip_hwc.reshape(L // 2, 2, Co)
    m1 = jnp.max(p1, axis=1)
    a1 = jnp.sum(p1, axis=1)
    p2m = m1.reshape(32, 2, 32, Co)
    p2a = a1.reshape(32, 2, 32, Co)
    mp = jnp.max(p2m, axis=1).reshape(1024, Co)
    ap = (jnp.sum(p2a, axis=1) * 0.25).reshape(1024, Co)

    cat = jnp.concatenate([mp, ap], axis=1)
    g = jax.nn.sigmoid(_dot_nt(cat, gw_ref[...]) + gb_ref[...])
    down_hwc = g * mp + (1.0 - g) * ap
    down_ref[0] = jnp.transpose(down_hwc)


def kernel(x, in_proj_w, conv1d_w, conv1d_b, x_proj_w, dt_proj_w, dt_proj_b,
           A_log, Dp, out_proj_w, pw_w, pw_b, bn_gamma, bn_beta,
           bn_mean, bn_var, gate_w, gate_b):
    B, C, H, W = x.shape
    L = H * W
    Di = in_proj_w.shape[0] // 2
    Co = pw_w.shape[0]

    x_seq = x.reshape(B, C, L).transpose(0, 2, 1)
    conv_w3 = conv1d_w[:, 0, :].T
    at = (-jnp.exp(A_log)).T
    dt_w = x_proj_w[:DT_RANK].T @ dt_proj_w.T
    bc_w = x_proj_w[DT_RANK:]
    pw_taps = pw_w.transpose(2, 3, 0, 1).reshape(9, Co, C)
    bn_scale = bn_gamma * lax.rsqrt(bn_var + BN_EPS)
    bn_bias = (pw_b - bn_mean) * bn_scale + bn_beta
    gate_w2 = gate_w[:, :, 0, 0]

    row = lambda v: v.reshape(1, -1)

    def _w(shape):
        nd = len(shape)
        return pl.BlockSpec(shape, lambda b, _n=nd: (0,) * _n)

    skip_flat, down_flat = pl.pallas_call(
        _block,
        grid=(B,),
        in_specs=[
            pl.BlockSpec((1, L, C), lambda b: (b, 0, 0)),
            _w((2 * Di, C)),
            _w((3, Di)),
            _w((1, Di)),
            _w((Di, Di)),
            _w((1, Di)),
            _w((2 * D_STATE, Di)),
            _w((D_STATE, Di)),
            _w((1, Di)),
            _w((C, Di)),
            _w((9, Co, C)),
            _w((1, Co)),
            _w((1, Co)),
            _w((Co, 2 * Co)),
            _w((1, Co)),
        ],
        out_specs=[
            pl.BlockSpec((1, Co, L), lambda b: (b, 0, 0)),
            pl.BlockSpec((1, Co, L // 4), lambda b: (b, 0, 0)),
        ],
        out_shape=[
            jax.ShapeDtypeStruct((B, Co, L), _F32),
            jax.ShapeDtypeStruct((B, Co, L // 4), _F32),
        ],
        scratch_shapes=[
            pltpu.VMEM((L, Di), _F32),
            pltpu.VMEM((L, Di), _F32),
            pltpu.VMEM((2 * D_STATE, L), _F32),
            pltpu.VMEM((L, 1, Di), _F32),
        ],
        compiler_params=pltpu.CompilerParams(
            dimension_semantics=("parallel",),
            vmem_limit_bytes=50 * 1024 * 1024,
        ),
    )(x_seq, in_proj_w, conv_w3, row(conv1d_b), dt_w, row(dt_proj_b),
      bc_w, at, row(Dp), out_proj_w, pw_taps, row(bn_scale), row(bn_bias),
      gate_w2, row(gate_b))

    skip = skip_flat.reshape(B, Co, H, W)
    down = down_flat.reshape(B, Co, H // 2, W // 2)
    return (down, skip)


# speedup vs baseline: 88.9156x; 1.1228x over previous
import jax
import jax.numpy as jnp
from jax import lax
from jax.experimental import pallas as pl
from jax.experimental.pallas import tpu as pltpu

D_STATE = 16
DT_RANK = 4
BN_EPS = 1e-5

_F32 = jnp.float32
_DIMS_NT = (((1,), (1,)), ((), ()))
_DIMS_NN = (((1,), (0,)), ((), ()))


def _dot_nt(a, b):
    return lax.dot_general(a, b, _DIMS_NT, preferred_element_type=_F32)


def _dot_nn(a, b):
    return lax.dot_general(a, b, _DIMS_NN, preferred_element_type=_F32)


def _block(seq_ref, wi_ref, cw_ref, cb_ref, dtw_ref, dtb_ref, bcw_ref,
           at_ref, dp_ref, wo_ref, pw_ref, bns_ref, bnb_ref, gw_ref, gb_ref,
           skip_ref, down_ref, dt_s, du_s, bct_s, ys_s):
    L, C = seq_ref.shape[1], seq_ref.shape[2]
    Di = dp_ref.shape[1]
    Wd = 64
    H_IMG = L // Wd
    seq = seq_ref[0]

    xz = _dot_nt(seq, wi_ref[...])
    u0 = xz[:, :Di]
    z = xz[:, Di:]

    um1 = jnp.concatenate([jnp.zeros((1, Di), _F32), u0[:-1]], axis=0)
    um2 = jnp.concatenate([jnp.zeros((2, Di), _F32), u0[:-2]], axis=0)
    uc = (u0 * cw_ref[2:3, :] + um1 * cw_ref[1:2, :] + um2 * cw_ref[0:1, :]
          + cb_ref[...])
    u = uc * jax.nn.sigmoid(uc)

    dt = jax.nn.softplus(_dot_nn(u, dtw_ref[...]) + dtb_ref[...])
    bct = _dot_nt(bcw_ref[...], u)

    dt_s[...] = dt
    du_s[...] = dt * u
    bct_s[...] = bct

    at = at_ref[...]
    TB = 128

    def body(i, h):
        t0 = pl.multiple_of(i * TB, TB)
        dtb = dt_s[pl.ds(t0, TB), :]
        dub = du_s[pl.ds(t0, TB), :]
        bcb = bct_s[:, pl.ds(t0, TB)]
        for jj in range(TB // 16):
            dA3 = jnp.exp(
                dtb[jj * 16:(jj + 1) * 16].reshape(16, 1, Di) * at[None])
            for j2 in range(16):
                j = jj * 16 + j2
                bc = jnp.broadcast_to(bcb[0:D_STATE, j:j + 1], (D_STATE, Di))
                cc = jnp.broadcast_to(
                    bcb[D_STATE:2 * D_STATE, j:j + 1], (D_STATE, Di))
                h = dA3[j2] * h + dub[j:j + 1, :] * bc
                ys_s[pl.ds(t0 + j, 1), :] = jnp.sum(
                    h * cc, axis=0, keepdims=True)
        return h

    lax.fori_loop(0, L // TB, body, jnp.zeros((D_STATE, Di), _F32))

    ys = ys_s[...]
    y = (ys + dp_ref[...] * u) * (z * jax.nn.sigmoid(z))
    res = seq + _dot_nt(y, wo_ref[...])

    col = lax.broadcasted_iota(jnp.int32, (L, 1), 0) % Wd
    acc = None
    k = 0
    for ky in (-1, 0, 1):
        for kx in (-1, 0, 1):
            s = ky * Wd + kx
            if s > 0:
                tap = jnp.concatenate(
                    [res[s:], jnp.zeros((s, C), _F32)], axis=0)
            elif s < 0:
                tap = jnp.concatenate(
                    [jnp.zeros((-s, C), _F32), res[:s]], axis=0)
            else:
                tap = res
            if kx == 1:
                tap = jnp.where(col != Wd - 1, tap, 0.0)
            elif kx == -1:
                tap = jnp.where(col != 0, tap, 0.0)
            part = _dot_nt(tap, pw_ref[k])
            acc = part if acc is None else acc + part
            k += 1

    skip_hwc = jnp.maximum(acc * bns_ref[...] + bnb_ref[...], 0.0)
    skip_ref[0] = jnp.transpose(skip_hwc)

    Co = skip_hwc.shape[1]
    y3 = skip_hwc.reshape(H_IMG // 2, 2, Wd, Co)
    my = jnp.maximum(y3[:, 0], y3[:, 1]).reshape(L // 2, Co)
    ay = (y3[:, 0] + y3[:, 1]).reshape(L // 2, Co)
    pm = my.reshape(L // 4, 2, Co)
    pa = ay.reshape(L // 4, 2, Co)
    mp = jnp.max(pm, axis=1)
    ap = jnp.sum(pa, axis=1) * 0.25

    cat = jnp.concatenate([mp, ap], axis=1)
    g = jax.nn.sigmoid(_dot_nt(cat, gw_ref[...]) + gb_ref[...])
    down_hwc = g * mp + (1.0 - g) * ap
    down_ref[0] = jnp.transpose(down_hwc)


def kernel(x, in_proj_w, conv1d_w, conv1d_b, x_proj_w, dt_proj_w, dt_proj_b,
           A_log, Dp, out_proj_w, pw_w, pw_b, bn_gamma, bn_beta,
           bn_mean, bn_var, gate_w, gate_b):
    B, C, H, W = x.shape
    L = H * W
    Di = in_proj_w.shape[0] // 2
    Co = pw_w.shape[0]

    x_seq = x.reshape(B, C, L).transpose(0, 2, 1)
    conv_w3 = conv1d_w[:, 0, :].T
    at = (-jnp.exp(A_log)).T
    dt_w = x_proj_w[:DT_RANK].T @ dt_proj_w.T
    bc_w = x_proj_w[DT_RANK:]
    pw_taps = pw_w.transpose(2, 3, 0, 1).reshape(9, Co, C)
    bn_scale = bn_gamma * lax.rsqrt(bn_var + BN_EPS)
    bn_bias = (pw_b - bn_mean) * bn_scale + bn_beta
    gate_w2 = gate_w[:, :, 0, 0]

    row = lambda v: v.reshape(1, -1)

    def _w(shape):
        nd = len(shape)
        return pl.BlockSpec(shape, lambda b, _n=nd: (0,) * _n)

    skip_flat, down_flat = pl.pallas_call(
        _block,
        grid=(B,),
        in_specs=[
            pl.BlockSpec((1, L, C), lambda b: (b, 0, 0)),
            _w((2 * Di, C)),
            _w((3, Di)),
            _w((1, Di)),
            _w((Di, Di)),
            _w((1, Di)),
            _w((2 * D_STATE, Di)),
            _w((D_STATE, Di)),
            _w((1, Di)),
            _w((C, Di)),
            _w((9, Co, C)),
            _w((1, Co)),
            _w((1, Co)),
            _w((Co, 2 * Co)),
            _w((1, Co)),
        ],
        out_specs=[
            pl.BlockSpec((1, Co, L), lambda b: (b, 0, 0)),
            pl.BlockSpec((1, Co, L // 4), lambda b: (b, 0, 0)),
        ],
        out_shape=[
            jax.ShapeDtypeStruct((B, Co, L), _F32),
            jax.ShapeDtypeStruct((B, Co, L // 4), _F32),
        ],
        scratch_shapes=[
            pltpu.VMEM((L, Di), _F32),
            pltpu.VMEM((L, Di), _F32),
            pltpu.VMEM((2 * D_STATE, L), _F32),
            pltpu.VMEM((L, Di), _F32),
        ],
        compiler_params=pltpu.CompilerParams(
            dimension_semantics=("parallel",),
            vmem_limit_bytes=50 * 1024 * 1024,
        ),
    )(x_seq, in_proj_w, conv_w3, row(conv1d_b), dt_w, row(dt_proj_b),
      bc_w, at, row(Dp), out_proj_w, pw_taps, row(bn_scale), row(bn_bias),
      gate_w2, row(gate_b))

    skip = skip_flat.reshape(B, Co, H, W)
    down = down_flat.reshape(B, Co, H // 2, W // 2)
    return (down, skip)


# speedup vs baseline: 93.5945x; 1.0526x over previous
import jax
import jax.numpy as jnp
from jax import lax
from jax.experimental import pallas as pl
from jax.experimental.pallas import tpu as pltpu

D_STATE = 16
DT_RANK = 4
BN_EPS = 1e-5
G = 2

_F32 = jnp.float32
_DIMS_NT = (((1,), (1,)), ((), ()))
_DIMS_NN = (((1,), (0,)), ((), ()))


def _dot_nt(a, b):
    return lax.dot_general(a, b, _DIMS_NT, preferred_element_type=_F32)


def _dot_nn(a, b):
    return lax.dot_general(a, b, _DIMS_NN, preferred_element_type=_F32)


def _block(seq_ref, wi_ref, cw_ref, cb_ref, dtw_ref, dtb_ref, bcw_ref,
           at_ref, dp_ref, wo_ref, pw_ref, bns_ref, bnb_ref, gw_ref, gb_ref,
           skip_ref, down_ref, dt_s, du_s, bct_s, ys_s):
    L, C = seq_ref.shape[1], seq_ref.shape[2]
    Di = dp_ref.shape[1]
    Wd = 64
    H_IMG = L // Wd
    GL = G * L
    seq = seq_ref[...].reshape(GL, C)

    xz = _dot_nt(seq, wi_ref[...])
    u0 = xz[:, :Di]
    z = xz[:, Di:]

    row = lax.broadcasted_iota(jnp.int32, (GL, 1), 0) % L
    um1 = jnp.concatenate([jnp.zeros((1, Di), _F32), u0[:-1]], axis=0)
    um2 = jnp.concatenate([jnp.zeros((2, Di), _F32), u0[:-2]], axis=0)
    um1 = jnp.where(row >= 1, um1, 0.0)
    um2 = jnp.where(row >= 2, um2, 0.0)
    uc = (u0 * cw_ref[2:3, :] + um1 * cw_ref[1:2, :] + um2 * cw_ref[0:1, :]
          + cb_ref[...])
    u = uc * jax.nn.sigmoid(uc)

    dt = jax.nn.softplus(_dot_nn(u, dtw_ref[...]) + dtb_ref[...])
    bct = _dot_nt(bcw_ref[...], u)

    dt_s[...] = dt
    du_s[...] = dt * u
    bct_s[...] = bct

    at = at_ref[...]
    TB = 128

    def body(i, hh):
        t0 = pl.multiple_of(i * TB, TB)
        hs = list(hh)
        dtb = [dt_s[pl.ds(b * L + t0, TB), :] for b in range(G)]
        dub = [du_s[pl.ds(b * L + t0, TB), :] for b in range(G)]
        bcb = [bct_s[:, pl.ds(b * L + t0, TB)] for b in range(G)]
        for jj in range(TB // 16):
            dA3 = [jnp.exp(
                dtb[b][jj * 16:(jj + 1) * 16].reshape(16, 1, Di) * at[None])
                for b in range(G)]
            for j2 in range(16):
                j = jj * 16 + j2
                for b in range(G):
                    bc = jnp.broadcast_to(
                        bcb[b][0:D_STATE, j:j + 1], (D_STATE, Di))
                    cc = jnp.broadcast_to(
                        bcb[b][D_STATE:2 * D_STATE, j:j + 1], (D_STATE, Di))
                    hs[b] = dA3[b][j2] * hs[b] + dub[b][j:j + 1, :] * bc
                    ys_s[pl.ds(b * L + t0 + j, 1), :] = jnp.sum(
                        hs[b] * cc, axis=0, keepdims=True)
        return tuple(hs)

    lax.fori_loop(0, L // TB, body,
                  tuple(jnp.zeros((D_STATE, Di), _F32) for _ in range(G)))

    y = (ys_s[...] + dp_ref[...] * u) * (z * jax.nn.sigmoid(z))
    res = seq + _dot_nt(y, wo_ref[...])

    col = lax.broadcasted_iota(jnp.int32, (L, 1), 0) % Wd
    Co = bns_ref.shape[1]
    for b in range(G):
        resb = res[b * L:(b + 1) * L]
        acc = None
        k = 0
        for ky in (-1, 0, 1):
            for kx in (-1, 0, 1):
                s = ky * Wd + kx
                if s > 0:
                    tap = jnp.concatenate(
                        [resb[s:], jnp.zeros((s, C), _F32)], axis=0)
                elif s < 0:
                    tap = jnp.concatenate(
                        [jnp.zeros((-s, C), _F32), resb[:s]], axis=0)
                else:
                    tap = resb
                if kx == 1:
                    tap = jnp.where(col != Wd - 1, tap, 0.0)
                elif kx == -1:
                    tap = jnp.where(col != 0, tap, 0.0)
                part = _dot_nt(tap, pw_ref[k])
                acc = part if acc is None else acc + part
                k += 1

        skip_hwc = jnp.maximum(acc * bns_ref[...] + bnb_ref[...], 0.0)
        skip_ref[b] = jnp.transpose(skip_hwc)

        y3 = skip_hwc.reshape(H_IMG // 2, 2, Wd, Co)
        my = jnp.maximum(y3[:, 0], y3[:, 1]).reshape(L // 2, Co)
        ay = (y3[:, 0] + y3[:, 1]).reshape(L // 2, Co)
        mp = jnp.max(my.reshape(L // 4, 2, Co), axis=1)
        ap = jnp.sum(ay.reshape(L // 4, 2, Co), axis=1) * 0.25

        cat = jnp.concatenate([mp, ap], axis=1)
        g = jax.nn.sigmoid(_dot_nt(cat, gw_ref[...]) + gb_ref[...])
        down_hwc = g * mp + (1.0 - g) * ap
        down_ref[b] = jnp.transpose(down_hwc)


def kernel(x, in_proj_w, conv1d_w, conv1d_b, x_proj_w, dt_proj_w, dt_proj_b,
           A_log, Dp, out_proj_w, pw_w, pw_b, bn_gamma, bn_beta,
           bn_mean, bn_var, gate_w, gate_b):
    B, C, H, W = x.shape
    L = H * W
    Di = in_proj_w.shape[0] // 2
    Co = pw_w.shape[0]

    x_seq = x.reshape(B, C, L).transpose(0, 2, 1)
    conv_w3 = conv1d_w[:, 0, :].T
    at = (-jnp.exp(A_log)).T
    dt_w = x_proj_w[:DT_RANK].T @ dt_proj_w.T
    bc_w = x_proj_w[DT_RANK:]
    pw_taps = pw_w.transpose(2, 3, 0, 1).reshape(9, Co, C)
    bn_scale = bn_gamma * lax.rsqrt(bn_var + BN_EPS)
    bn_bias = (pw_b - bn_mean) * bn_scale + bn_beta
    gate_w2 = gate_w[:, :, 0, 0]

    rowv = lambda v: v.reshape(1, -1)

    def _w(shape):
        nd = len(shape)
        return pl.BlockSpec(shape, lambda g, _n=nd: (0,) * _n)

    skip_flat, down_flat = pl.pallas_call(
        _block,
        grid=(B // G,),
        in_specs=[
            pl.BlockSpec((G, L, C), lambda g: (g, 0, 0)),
            _w((2 * Di, C)),
            _w((3, Di)),
            _w((1, Di)),
            _w((Di, Di)),
            _w((1, Di)),
            _w((2 * D_STATE, Di)),
            _w((D_STATE, Di)),
            _w((1, Di)),
            _w((C, Di)),
            _w((9, Co, C)),
            _w((1, Co)),
            _w((1, Co)),
            _w((Co, 2 * Co)),
            _w((1, Co)),
        ],
        out_specs=[
            pl.BlockSpec((G, Co, L), lambda g: (g, 0, 0)),
            pl.BlockSpec((G, Co, L // 4), lambda g: (g, 0, 0)),
        ],
        out_shape=[
            jax.ShapeDtypeStruct((B, Co, L), _F32),
            jax.ShapeDtypeStruct((B, Co, L // 4), _F32),
        ],
        scratch_shapes=[
            pltpu.VMEM((G * L, Di), _F32),
            pltpu.VMEM((G * L, Di), _F32),
            pltpu.VMEM((2 * D_STATE, G * L), _F32),
            pltpu.VMEM((G * L, Di), _F32),
        ],
        compiler_params=pltpu.CompilerParams(
            dimension_semantics=("parallel",),
            vmem_limit_bytes=56 * 1024 * 1024,
        ),
    )(x_seq, in_proj_w, conv_w3, rowv(conv1d_b), dt_w, rowv(dt_proj_b),
      bc_w, at, rowv(Dp), out_proj_w, pw_taps, rowv(bn_scale),
      rowv(bn_bias), gate_w2, rowv(gate_b))

    skip = skip_flat.reshape(B, Co, H, W)
    down = down_flat.reshape(B, Co, H // 2, W // 2)
    return (down, skip)


# speedup vs baseline: 96.8644x; 1.0349x over previous
import jax
import jax.numpy as jnp
from jax import lax
from jax.experimental import pallas as pl
from jax.experimental.pallas import tpu as pltpu

D_STATE = 16
DT_RANK = 4
BN_EPS = 1e-5
G = 4

_F32 = jnp.float32
_DIMS_NT = (((1,), (1,)), ((), ()))
_DIMS_NN = (((1,), (0,)), ((), ()))


def _dot_nt(a, b):
    return lax.dot_general(a, b, _DIMS_NT, preferred_element_type=_F32)


def _dot_nn(a, b):
    return lax.dot_general(a, b, _DIMS_NN, preferred_element_type=_F32)


def _block(seq_ref, wi_ref, cw_ref, cb_ref, dtw_ref, dtb_ref, bcw_ref,
           at_ref, dp_ref, wo_ref, pw_ref, bns_ref, bnb_ref, gw_ref, gb_ref,
           skip_ref, down_ref, dt_s, u_s, bct_s):
    L, C = seq_ref.shape[1], seq_ref.shape[2]
    Di = dp_ref.shape[1]
    Wd = 64
    H_IMG = L // Wd
    GL = G * L
    seq = seq_ref[...].reshape(GL, C)

    xz = _dot_nt(seq, wi_ref[...])
    u0 = xz[:, :Di]
    z = xz[:, Di:]

    row = lax.broadcasted_iota(jnp.int32, (GL, 1), 0) % L
    um1 = jnp.concatenate([jnp.zeros((1, Di), _F32), u0[:-1]], axis=0)
    um2 = jnp.concatenate([jnp.zeros((2, Di), _F32), u0[:-2]], axis=0)
    um1 = jnp.where(row >= 1, um1, 0.0)
    um2 = jnp.where(row >= 2, um2, 0.0)
    uc = (u0 * cw_ref[2:3, :] + um1 * cw_ref[1:2, :] + um2 * cw_ref[0:1, :]
          + cb_ref[...])
    u = uc * jax.nn.sigmoid(uc)

    dt = jax.nn.softplus(_dot_nn(u, dtw_ref[...]) + dtb_ref[...])
    bct = _dot_nt(bcw_ref[...], u)

    dt_s[...] = dt
    u_s[...] = u
    bct_s[...] = bct

    at = at_ref[...]
    TB = 128

    def body(i, hh):
        t0 = pl.multiple_of(i * TB, TB)
        hs = list(hh)
        dtb = [dt_s[pl.ds(b * L + t0, TB), :] for b in range(G)]
        dub = [dtb[b] * u_s[pl.ds(b * L + t0, TB), :] for b in range(G)]
        bcb = [bct_s[:, pl.ds(b * L + t0, TB)] for b in range(G)]
        for jj in range(TB // 16):
            dA3 = [jnp.exp(
                dtb[b][jj * 16:(jj + 1) * 16].reshape(16, 1, Di) * at[None])
                for b in range(G)]
            for j2 in range(16):
                j = jj * 16 + j2
                for b in range(G):
                    bc = jnp.broadcast_to(
                        bcb[b][0:D_STATE, j:j + 1], (D_STATE, Di))
                    cc = jnp.broadcast_to(
                        bcb[b][D_STATE:2 * D_STATE, j:j + 1], (D_STATE, Di))
                    hs[b] = dA3[b][j2] * hs[b] + dub[b][j:j + 1, :] * bc
                    dt_s[pl.ds(b * L + t0 + j, 1), :] = jnp.sum(
                        hs[b] * cc, axis=0, keepdims=True)
        return tuple(hs)

    lax.fori_loop(0, L // TB, body,
                  tuple(jnp.zeros((D_STATE, Di), _F32) for _ in range(G)))

    y = (dt_s[...] + dp_ref[...] * u_s[...]) * (z * jax.nn.sigmoid(z))
    res = seq + _dot_nt(y, wo_ref[...])

    col = lax.broadcasted_iota(jnp.int32, (L, 1), 0) % Wd
    Co = bns_ref.shape[1]
    for b in range(G):
        resb = res[b * L:(b + 1) * L]
        acc = None
        k = 0
        for ky in (-1, 0, 1):
            for kx in (-1, 0, 1):
                s = ky * Wd + kx
                if s > 0:
                    tap = jnp.concatenate(
                        [resb[s:], jnp.zeros((s, C), _F32)], axis=0)
                elif s < 0:
                    tap = jnp.concatenate(
                        [jnp.zeros((-s, C), _F32), resb[:s]], axis=0)
                else:
                    tap = resb
                if kx == 1:
                    tap = jnp.where(col != Wd - 1, tap, 0.0)
                elif kx == -1:
                    tap = jnp.where(col != 0, tap, 0.0)
                part = _dot_nt(tap, pw_ref[k])
                acc = part if acc is None else acc + part
                k += 1

        skip_hwc = jnp.maximum(acc * bns_ref[...] + bnb_ref[...], 0.0)
        skip_ref[b] = jnp.transpose(skip_hwc)

        y3 = skip_hwc.reshape(H_IMG // 2, 2, Wd, Co)
        my = jnp.maximum(y3[:, 0], y3[:, 1]).reshape(L // 2, Co)
        ay = (y3[:, 0] + y3[:, 1]).reshape(L // 2, Co)
        mp = jnp.max(my.reshape(L // 4, 2, Co), axis=1)
        ap = jnp.sum(ay.reshape(L // 4, 2, Co), axis=1) * 0.25

        cat = jnp.concatenate([mp, ap], axis=1)
        g = jax.nn.sigmoid(_dot_nt(cat, gw_ref[...]) + gb_ref[...])
        down_hwc = g * mp + (1.0 - g) * ap
        down_ref[b] = jnp.transpose(down_hwc)


def kernel(x, in_proj_w, conv1d_w, conv1d_b, x_proj_w, dt_proj_w, dt_proj_b,
           A_log, Dp, out_proj_w, pw_w, pw_b, bn_gamma, bn_beta,
           bn_mean, bn_var, gate_w, gate_b):
    B, C, H, W = x.shape
    L = H * W
    Di = in_proj_w.shape[0] // 2
    Co = pw_w.shape[0]

    x_seq = x.reshape(B, C, L).transpose(0, 2, 1)
    conv_w3 = conv1d_w[:, 0, :].T
    at = (-jnp.exp(A_log)).T
    dt_w = x_proj_w[:DT_RANK].T @ dt_proj_w.T
    bc_w = x_proj_w[DT_RANK:]
    pw_taps = pw_w.transpose(2, 3, 0, 1).reshape(9, Co, C)
    bn_scale = bn_gamma * lax.rsqrt(bn_var + BN_EPS)
    bn_bias = (pw_b - bn_mean) * bn_scale + bn_beta
    gate_w2 = gate_w[:, :, 0, 0]

    rowv = lambda v: v.reshape(1, -1)

    def _w(shape):
        nd = len(shape)
        return pl.BlockSpec(shape, lambda g, _n=nd: (0,) * _n)

    skip_flat, down_flat = pl.pallas_call(
        _block,
        grid=(B // G,),
        in_specs=[
            pl.BlockSpec((G, L, C), lambda g: (g, 0, 0)),
            _w((2 * Di, C)),
            _w((3, Di)),
            _w((1, Di)),
            _w((Di, Di)),
            _w((1, Di)),
            _w((2 * D_STATE, Di)),
            _w((D_STATE, Di)),
            _w((1, Di)),
            _w((C, Di)),
            _w((9, Co, C)),
            _w((1, Co)),
            _w((1, Co)),
            _w((Co, 2 * Co)),
            _w((1, Co)),
        ],
        out_specs=[
            pl.BlockSpec((G, Co, L), lambda g: (g, 0, 0)),
            pl.BlockSpec((G, Co, L // 4), lambda g: (g, 0, 0)),
        ],
        out_shape=[
            jax.ShapeDtypeStruct((B, Co, L), _F32),
            jax.ShapeDtypeStruct((B, Co, L // 4), _F32),
        ],
        scratch_shapes=[
            pltpu.VMEM((G * L, Di), _F32),
            pltpu.VMEM((G * L, Di), _F32),
            pltpu.VMEM((2 * D_STATE, G * L), _F32),
        ],
        compiler_params=pltpu.CompilerParams(
            dimension_semantics=("parallel",),
            vmem_limit_bytes=61 * 1024 * 1024,
        ),
    )(x_seq, in_proj_w, conv_w3, rowv(conv1d_b), dt_w, rowv(dt_proj_b),
      bc_w, at, rowv(Dp), out_proj_w, pw_taps, rowv(bn_scale),
      rowv(bn_bias), gate_w2, rowv(gate_b))

    skip = skip_flat.reshape(B, Co, H, W)
    down = down_flat.reshape(B, Co, H // 2, W // 2)
    return (down, skip)
